# baseline (speedup 1.0000x reference)
import jax
import jax.numpy as jnp
from jax import lax
from jax.experimental import pallas as pl
from jax.experimental.pallas import tpu as pltpu
from jax.experimental.pallas import tpu_sc as plsc

B = 16384
D = 64
W = 2 * D
NC = 2
NS = 16
NW = NC * NS
BPW = B // NW
CHUNK = 128
NCHUNK = BPW // CHUNK


def _sc_gather_body(uidx_hbm, iidx_hbm, ug_hbm, ig_hbm, um_hbm, im_hbm,
                    ug_out, ig_out, um_out, im_out,
                    uidx_v, iidx_v, buf0, buf1, sem0, sem1):
  wid = lax.axis_index("s") * NC + lax.axis_index("c")
  base = wid * BPW
  pltpu.sync_copy(uidx_hbm.at[pl.ds(base, BPW)], uidx_v)
  pltpu.sync_copy(iidx_hbm.at[pl.ds(base, BPW)], iidx_v)

  bufs = (buf0, buf1)
  sems = (sem0, sem1)
  plan = []
  for tab, idx_v, out in ((ug_hbm, uidx_v, ug_out),
                          (ig_hbm, iidx_v, ig_out),
                          (um_hbm, uidx_v, um_out),
                          (im_hbm, iidx_v, im_out)):
    for j in range(NCHUNK):
      plan.append((tab, idx_v, j * CHUNK, out))

  pending = [None, None]
  for t, (tab, idx_v, off, out) in enumerate(plan):
    s = t % 2
    if pending[s] is not None:
      cp, pout, poff = pending[s]
      cp.wait()
      pltpu.sync_copy(bufs[s], pout.at[pl.ds(base + poff, CHUNK)])
    cp = pltpu.async_copy(tab.at[idx_v.at[pl.ds(off, CHUNK)]], bufs[s],
                          sems[s])
    pending[s] = (cp, out, off)
  for s in range(2):
    cp, pout, poff = pending[s]
    cp.wait()
    pltpu.sync_copy(bufs[s], pout.at[pl.ds(base + poff, CHUNK)])


_sc_gather = pl.kernel(
    _sc_gather_body,
    out_type=[jax.ShapeDtypeStruct((B, W), jnp.float32)] * 4,
    mesh=plsc.VectorSubcoreMesh(core_axis_name="c", subcore_axis_name="s"),
    scratch_types=[
        pltpu.VMEM((BPW,), jnp.int32),
        pltpu.VMEM((BPW,), jnp.int32),
        pltpu.VMEM((CHUNK, W), jnp.float32),
        pltpu.VMEM((CHUNK, W), jnp.float32),
        pltpu.SemaphoreType.DMA,
        pltpu.SemaphoreType.DMA,
    ],
)

BLK = 2048


def _dense_body(ugw_ref, igw_ref, umw_ref, imw_ref, pu_ref, pi_ref,
                w1t_ref, b1_ref, wp_ref, bp_ref, out_ref):
  pu = pu_ref[...]
  pi = pi_ref[...]

  def sel(w_ref, p):
    return w_ref[:, :D] * (1.0 - p) + w_ref[:, D:] * p

  gmf = sel(ugw_ref, pu) * sel(igw_ref, pi)
  h = jnp.dot(sel(umw_ref, pu), w1t_ref[:D, :],
              preferred_element_type=jnp.float32)
  h = h + jnp.dot(sel(imw_ref, pi), w1t_ref[D:, :],
                  preferred_element_type=jnp.float32)
  h = jnp.maximum(h + b1_ref[...], 0.0)
  pred = jnp.sum(gmf * wp_ref[:, :D], axis=1)
  pred = pred + jnp.sum(h * wp_ref[:, D:], axis=1)
  out_ref[...] = pred + bp_ref[0, 0]


def _dense_call(ugw, igw, umw, imw, pu, pi, w1t, b1_2d, wp, bp_2d):
  grid = (B // BLK,)
  row_spec = pl.BlockSpec((BLK, W), lambda i: (i, 0))
  par_spec = pl.BlockSpec((BLK, 1), lambda i: (i, 0))
  return pl.pallas_call(
      _dense_body,
      grid=grid,
      in_specs=[
          row_spec, row_spec, row_spec, row_spec,
          par_spec, par_spec,
          pl.BlockSpec((W, D), lambda i: (0, 0)),
          pl.BlockSpec((1, D), lambda i: (0, 0)),
          pl.BlockSpec((1, W), lambda i: (0, 0)),
          pl.BlockSpec((1, 1), lambda i: (0, 0)),
      ],
      out_specs=pl.BlockSpec((BLK,), lambda i: (i,)),
      out_shape=jax.ShapeDtypeStruct((B,), jnp.float32),
  )(ugw, igw, umw, imw, pu, pi, w1t, b1_2d, wp, bp_2d)


def kernel(user_indices, item_indices, user_gmf_table, item_gmf_table,
           user_mlp_table, item_mlp_table, W1, b1, Wp, bp):
  uidx = user_indices.astype(jnp.int32)
  iidx = item_indices.astype(jnp.int32)
  ug2 = user_gmf_table.reshape(-1, W)
  ig2 = item_gmf_table.reshape(-1, W)
  um2 = user_mlp_table.reshape(-1, W)
  im2 = item_mlp_table.reshape(-1, W)
  ugw, igw, umw, imw = _sc_gather(
      uidx >> 1, iidx >> 1, ug2, ig2, um2, im2)
  pu = (uidx & 1).astype(jnp.float32).reshape(B, 1)
  pi = (iidx & 1).astype(jnp.float32).reshape(B, 1)
  w1t = W1.T
  return _dense_call(ugw, igw, umw, imw, pu, pi, w1t, b1.reshape(1, D), Wp,
                     bp.reshape(1, 1))


# speedup vs baseline: 1.3175x; 1.3175x over previous
import jax
import jax.numpy as jnp
from jax import lax
from jax.experimental import pallas as pl
from jax.experimental.pallas import tpu as pltpu
from jax.experimental.pallas import tpu_sc as plsc

B = 16384
D = 64
NC = 2
NS = 16
NW = NC * NS
BPW = B // NW
K = 16
HALF = BPW // 4


def _sc_gather_body(uidx_hbm, iidx_hbm, ug_hbm, ig_hbm, um_hbm, im_hbm,
                    ug_out, ig_out, um_out, im_out,
                    uidx_v, iidx_v, ug_buf, ig_buf, um_buf, im_buf, sem):
  wid = lax.axis_index("s") * NC + lax.axis_index("c")
  base = wid * BPW
  pltpu.sync_copy(uidx_hbm.at[pl.ds(base, BPW)], uidx_v)
  pltpu.sync_copy(iidx_hbm.at[pl.ds(base, BPW)], iidx_v)

  def make_wave(half):
    def wave(w, _):
      row0 = half * HALF + w * K
      brow0 = w * K
      uvec = uidx_v[pl.ds(row0, K)]
      ivec = iidx_v[pl.ds(row0, K)]
      cps = []
      for j in range(K):
        ru = uvec[j]
        ri = ivec[j]
        cps.append(pltpu.async_copy(
            ug_hbm.at[pl.ds(ru, 1), :], ug_buf.at[pl.ds(brow0 + j, 1), :],
            sem))
        cps.append(pltpu.async_copy(
            um_hbm.at[pl.ds(ru, 1), :], um_buf.at[pl.ds(brow0 + j, 1), :],
            sem))
        cps.append(pltpu.async_copy(
            ig_hbm.at[pl.ds(ri, 1), :], ig_buf.at[pl.ds(brow0 + j, 1), :],
            sem))
        cps.append(pltpu.async_copy(
            im_hbm.at[pl.ds(ri, 1), :], im_buf.at[pl.ds(brow0 + j, 1), :],
            sem))
      for cp in cps:
        cp.wait()
      return _
    return wave

  for half in range(4):
    lax.fori_loop(0, HALF // K, make_wave(half), 0)
    off = base + half * HALF
    pltpu.sync_copy(ug_buf, ug_out.at[pl.ds(off, HALF)])
    pltpu.sync_copy(ig_buf, ig_out.at[pl.ds(off, HALF)])
    pltpu.sync_copy(um_buf, um_out.at[pl.ds(off, HALF)])
    pltpu.sync_copy(im_buf, im_out.at[pl.ds(off, HALF)])


_sc_gather = pl.kernel(
    _sc_gather_body,
    out_type=[jax.ShapeDtypeStruct((B, D), jnp.float32)] * 4,
    mesh=plsc.VectorSubcoreMesh(core_axis_name="c", subcore_axis_name="s"),
    scratch_types=[
        pltpu.VMEM((BPW,), jnp.int32),
        pltpu.VMEM((BPW,), jnp.int32),
        pltpu.VMEM((HALF, D), jnp.float32),
        pltpu.VMEM((HALF, D), jnp.float32),
        pltpu.VMEM((HALF, D), jnp.float32),
        pltpu.VMEM((HALF, D), jnp.float32),
        pltpu.SemaphoreType.DMA,
    ],
)

BLK = 2048


def _dense_body(ug_ref, ig_ref, um_ref, im_ref, w1t_ref, b1_ref, wp_ref,
                bp_ref, out_ref):
  gmf = ug_ref[...] * ig_ref[...]
  h = jnp.dot(um_ref[...], w1t_ref[:D, :], preferred_element_type=jnp.float32)
  h = h + jnp.dot(im_ref[...], w1t_ref[D:, :],
                  preferred_element_type=jnp.float32)
  h = jnp.maximum(h + b1_ref[...], 0.0)
  pred = jnp.sum(gmf * wp_ref[:, :D], axis=1)
  pred = pred + jnp.sum(h * wp_ref[:, D:], axis=1)
  out_ref[...] = pred + bp_ref[0, 0]


def _dense_call(ug, ig, um, im, w1t, b1_2d, wp, bp_2d):
  grid = (B // BLK,)
  row_spec = pl.BlockSpec((BLK, D), lambda i: (i, 0))
  return pl.pallas_call(
      _dense_body,
      grid=grid,
      in_specs=[
          row_spec, row_spec, row_spec, row_spec,
          pl.BlockSpec((2 * D, D), lambda i: (0, 0)),
          pl.BlockSpec((1, D), lambda i: (0, 0)),
          pl.BlockSpec((1, 2 * D), lambda i: (0, 0)),
          pl.BlockSpec((1, 1), lambda i: (0, 0)),
      ],
      out_specs=pl.BlockSpec((BLK,), lambda i: (i,)),
      out_shape=jax.ShapeDtypeStruct((B,), jnp.float32),
  )(ug, ig, um, im, w1t, b1_2d, wp, bp_2d)


def kernel(user_indices, item_indices, user_gmf_table, item_gmf_table,
           user_mlp_table, item_mlp_table, W1, b1, Wp, bp):
  ug, ig, um, im = _sc_gather(
      user_indices.astype(jnp.int32), item_indices.astype(jnp.int32),
      user_gmf_table, item_gmf_table, user_mlp_table, item_mlp_table)
  w1t = W1.T
  return _dense_call(ug, ig, um, im, w1t, b1.reshape(1, D), Wp,
                     bp.reshape(1, 1))


# speedup vs baseline: 1.4617x; 1.1095x over previous
import jax
import jax.numpy as jnp
from jax import lax
from jax.experimental import pallas as pl
from jax.experimental.pallas import tpu as pltpu
from jax.experimental.pallas import tpu_sc as plsc

B = 16384
D = 64
NC = 2
NS = 16
NW = NC * NS
BPW = B // NW
K = 16
PASS = BPW // 4


def _sc_gather_body(uidx_hbm, iidx_hbm, ug_hbm, ig_hbm, um_hbm, im_hbm,
                    ug_out, ig_out, um_out, im_out,
                    uidx_v, iidx_v, ug_buf, ig_buf, um_buf, im_buf, sem):
  wid = lax.axis_index("s") * NC + lax.axis_index("c")
  base = wid * BPW
  pltpu.sync_copy(uidx_hbm.at[pl.ds(base, BPW)], uidx_v)
  pltpu.sync_copy(iidx_hbm.at[pl.ds(base, BPW)], iidx_v)

  def make_wave(half):
    def wave(w, _):
      row0 = half * PASS + w * K
      brow0 = w * K
      uvec = uidx_v[pl.ds(row0, K)]
      ivec = iidx_v[pl.ds(row0, K)]
      for j in range(K):
        ru = uvec[j]
        ri = ivec[j]
        pltpu.async_copy(
            ug_hbm.at[pl.ds(ru, 1), :], ug_buf.at[pl.ds(brow0 + j, 1), :],
            sem)
        pltpu.async_copy(
            um_hbm.at[pl.ds(ru, 1), :], um_buf.at[pl.ds(brow0 + j, 1), :],
            sem)
        pltpu.async_copy(
            ig_hbm.at[pl.ds(ri, 1), :], ig_buf.at[pl.ds(brow0 + j, 1), :],
            sem)
        pltpu.async_copy(
            im_hbm.at[pl.ds(ri, 1), :], im_buf.at[pl.ds(brow0 + j, 1), :],
            sem)
      return _
    return wave

  for half in range(4):
    lax.fori_loop(0, PASS // K, make_wave(half), 0)
    for buf in (ug_buf, ig_buf, um_buf, im_buf):
      pltpu.make_async_copy(ug_hbm.at[pl.ds(0, PASS), :], buf, sem).wait()
    off = base + half * PASS
    pltpu.sync_copy(ug_buf, ug_out.at[pl.ds(off, PASS)])
    pltpu.sync_copy(ig_buf, ig_out.at[pl.ds(off, PASS)])
    pltpu.sync_copy(um_buf, um_out.at[pl.ds(off, PASS)])
    pltpu.sync_copy(im_buf, im_out.at[pl.ds(off, PASS)])


_sc_gather = pl.kernel(
    _sc_gather_body,
    out_type=[jax.ShapeDtypeStruct((B, D), jnp.float32)] * 4,
    mesh=plsc.VectorSubcoreMesh(core_axis_name="c", subcore_axis_name="s"),
    scratch_types=[
        pltpu.VMEM((BPW,), jnp.int32),
        pltpu.VMEM((BPW,), jnp.int32),
        pltpu.VMEM((PASS, D), jnp.float32),
        pltpu.VMEM((PASS, D), jnp.float32),
        pltpu.VMEM((PASS, D), jnp.float32),
        pltpu.VMEM((PASS, D), jnp.float32),
        pltpu.SemaphoreType.DMA,
    ],
)

BLK = 2048


def _dense_body(ug_ref, ig_ref, um_ref, im_ref, w1t_ref, b1_ref, wp_ref,
                bp_ref, out_ref):
  gmf = ug_ref[...] * ig_ref[...]
  h = jnp.dot(um_ref[...], w1t_ref[:D, :], preferred_element_type=jnp.float32)
  h = h + jnp.dot(im_ref[...], w1t_ref[D:, :],
                  preferred_element_type=jnp.float32)
  h = jnp.maximum(h + b1_ref[...], 0.0)
  pred = jnp.sum(gmf * wp_ref[:, :D], axis=1)
  pred = pred + jnp.sum(h * wp_ref[:, D:], axis=1)
  out_ref[...] = pred + bp_ref[0, 0]


def _dense_call(ug, ig, um, im, w1t, b1_2d, wp, bp_2d):
  grid = (B // BLK,)
  row_spec = pl.BlockSpec((BLK, D), lambda i: (i, 0))
  return pl.pallas_call(
      _dense_body,
      grid=grid,
      in_specs=[
          row_spec, row_spec, row_spec, row_spec,
          pl.BlockSpec((2 * D, D), lambda i: (0, 0)),
          pl.BlockSpec((1, D), lambda i: (0, 0)),
          pl.BlockSpec((1, 2 * D), lambda i: (0, 0)),
          pl.BlockSpec((1, 1), lambda i: (0, 0)),
      ],
      out_specs=pl.BlockSpec((BLK,), lambda i: (i,)),
      out_shape=jax.ShapeDtypeStruct((B,), jnp.float32),
  )(ug, ig, um, im, w1t, b1_2d, wp, bp_2d)


def kernel(user_indices, item_indices, user_gmf_table, item_gmf_table,
           user_mlp_table, item_mlp_table, W1, b1, Wp, bp):
  ug, ig, um, im = _sc_gather(
      user_indices.astype(jnp.int32), item_indices.astype(jnp.int32),
      user_gmf_table, item_gmf_table, user_mlp_table, item_mlp_table)
  w1t = W1.T
  return _dense_call(ug, ig, um, im, w1t, b1.reshape(1, D), Wp,
                     bp.reshape(1, 1))
